# Initial kernel scaffold; baseline (speedup 1.0000x reference)
import jax
import jax.numpy as jnp
from jax.experimental import pallas as pl


def kernel(data, iword_indicator, iword_numerals, ivectors_weight, gmm_posterior, iprototypes_embeddings):
    raise NotImplementedError("write your pallas kernel here")


# speedup vs baseline: 1.6856x; 1.6856x over previous
import functools

import jax
import jax.numpy as jnp
from jax import lax
from jax.experimental import pallas as pl
from jax.experimental.pallas import tpu as pltpu
from jax.experimental.pallas import tpu_sc as plsc

_B = 16384
_L = 50
_EMB = 64
_TOTAL = _B * _L
_NC = 2
_NS = 16
_NW = _NC * _NS
_PER_W = _TOTAL // _NW
_CHUNK = 128
_NCH = _PER_W // _CHUNK


@jax.jit
def _gather_call(table, idx3):
    mesh = plsc.VectorSubcoreMesh(core_axis_name="c", subcore_axis_name="s")

    @functools.partial(
        pl.kernel,
        mesh=mesh,
        out_type=jax.ShapeDtypeStruct((_TOTAL, _EMB), jnp.float32),
        scratch_types=[
            pltpu.VMEM((_NCH, _CHUNK), jnp.int32),
            pltpu.VMEM((_CHUNK, _EMB), jnp.float32),
            pltpu.SemaphoreType.DMA,
        ],
        compiler_params=pltpu.CompilerParams(use_tc_tiling_on_sc=False),
    )
    def k(table_hbm, idx_hbm, out_hbm, idx_v, rows_v, sem):
        wid = lax.axis_index("s") * _NC + lax.axis_index("c")
        base = wid * _PER_W
        pltpu.sync_copy(idx_hbm.at[wid], idx_v)

        def body(j, carry):
            pltpu.async_copy(table_hbm.at[idx_v.at[j]], rows_v, sem).wait()
            pltpu.sync_copy(rows_v, out_hbm.at[pl.ds(base + j * _CHUNK, _CHUNK)])
            return carry

        lax.fori_loop(0, _NCH, body, 0)

    return k(table, idx3)


def kernel(data, iword_indicator, iword_numerals, ivectors_weight,
           gmm_posterior, iprototypes_embeddings):
    idx3 = data.reshape(_NW, _NCH, _CHUNK)
    out = _gather_call(ivectors_weight, idx3)
    return out.reshape(_B, _L, _EMB)


# speedup vs baseline: 1.8326x; 1.0872x over previous
import functools

import jax
import jax.numpy as jnp
from jax import lax
from jax.experimental import pallas as pl
from jax.experimental.pallas import tpu as pltpu
from jax.experimental.pallas import tpu_sc as plsc

_B = 16384
_L = 50
_EMB = 64
_TOTAL = _B * _L
_NC = 2
_NS = 16
_NW = _NC * _NS
_PER_W = _TOTAL // _NW
_CHUNK = 512
_NCH = _PER_W // _CHUNK


@jax.jit
def _gather_call(table, idx3):
    mesh = plsc.VectorSubcoreMesh(core_axis_name="c", subcore_axis_name="s")

    @functools.partial(
        pl.kernel,
        mesh=mesh,
        out_type=jax.ShapeDtypeStruct((_TOTAL, _EMB), jnp.float32),
        scratch_types=[
            pltpu.VMEM((_NCH, _CHUNK), jnp.int32),
            pltpu.VMEM((_CHUNK, _EMB), jnp.float32),
            pltpu.SemaphoreType.DMA,
        ],
        compiler_params=pltpu.CompilerParams(use_tc_tiling_on_sc=False),
    )
    def k(table_hbm, idx_hbm, out_hbm, idx_v, rows_v, sem):
        wid = lax.axis_index("s") * _NC + lax.axis_index("c")
        base = wid * _PER_W
        pltpu.sync_copy(idx_hbm.at[wid], idx_v)

        def body(j, carry):
            pltpu.async_copy(table_hbm.at[idx_v.at[j]], rows_v, sem).wait()
            pltpu.sync_copy(rows_v, out_hbm.at[pl.ds(base + j * _CHUNK, _CHUNK)])
            return carry

        lax.fori_loop(0, _NCH, body, 0)

    return k(table, idx3)


def kernel(data, iword_indicator, iword_numerals, ivectors_weight,
           gmm_posterior, iprototypes_embeddings):
    idx3 = data.reshape(_NW, _NCH, _CHUNK)
    out = _gather_call(ivectors_weight, idx3)
    return out.reshape(_B, _L, _EMB)


# speedup vs baseline: 1.8767x; 1.0241x over previous
import functools

import jax
import jax.numpy as jnp
from jax import lax
from jax.experimental import pallas as pl
from jax.experimental.pallas import tpu as pltpu
from jax.experimental.pallas import tpu_sc as plsc

_B = 16384
_L = 50
_EMB = 64
_TOTAL = _B * _L
_NC = 2
_NS = 16
_NW = _NC * _NS
_PER_W = _TOTAL // _NW
_CHUNK = 512
_NCH = _PER_W // _CHUNK


@jax.jit
def _gather_call(table, idx3):
    mesh = plsc.VectorSubcoreMesh(core_axis_name="c", subcore_axis_name="s")

    @functools.partial(
        pl.kernel,
        mesh=mesh,
        out_type=jax.ShapeDtypeStruct((_TOTAL, _EMB), jnp.float32),
        scratch_types=[
            pltpu.VMEM((_NCH, _CHUNK), jnp.int32),
            pltpu.VMEM((_CHUNK, _EMB), jnp.float32),
            pltpu.VMEM((_CHUNK, _EMB), jnp.float32),
            pltpu.SemaphoreType.DMA,
            pltpu.SemaphoreType.DMA,
            pltpu.SemaphoreType.DMA,
            pltpu.SemaphoreType.DMA,
        ],
        compiler_params=pltpu.CompilerParams(use_tc_tiling_on_sc=False),
    )
    def k(table_hbm, idx_hbm, out_hbm, idx_v, rows0, rows1, g0, g1, o0, o1):
        wid = lax.axis_index("s") * _NC + lax.axis_index("c")
        base = wid * _PER_W
        rows = [rows0, rows1]
        gsem = [g0, g1]
        osem = [o0, o1]
        pltpu.sync_copy(idx_hbm.at[wid], idx_v)

        def start_gather(j, b):
            pltpu.async_copy(table_hbm.at[idx_v.at[j]], rows[b], gsem[b])

        def wait_gather(j, b):
            pltpu.make_async_copy(table_hbm.at[idx_v.at[j]], rows[b],
                                  gsem[b]).wait()

        def out_slice(j):
            return out_hbm.at[pl.ds(base + j * _CHUNK, _CHUNK)]

        def start_store(j, b):
            pltpu.async_copy(rows[b], out_slice(j), osem[b])

        def wait_store(j, b):
            pltpu.make_async_copy(rows[b], out_slice(j), osem[b]).wait()

        start_gather(0, 0)

        def body(i, carry):
            for b in range(2):
                j = 2 * i + b
                nxt = 1 - b

                @pl.when(j + 1 < _NCH)
                def _():
                    @pl.when(j >= 1)
                    def _():
                        wait_store(j - 1, nxt)
                    start_gather(j + 1, nxt)

                wait_gather(j, b)
                start_store(j, b)
            return carry

        lax.fori_loop(0, _NCH // 2, body, 0)
        wait_store(_NCH - 2, 0)
        wait_store(_NCH - 1, 1)

    return k(table, idx3)


def kernel(data, iword_indicator, iword_numerals, ivectors_weight,
           gmm_posterior, iprototypes_embeddings):
    idx3 = data.reshape(_NW, _NCH, _CHUNK)
    out = _gather_call(ivectors_weight, idx3)
    return out.reshape(_B, _L, _EMB)
